# Initial kernel scaffold; baseline (speedup 1.0000x reference)
import jax
import jax.numpy as jnp
from jax.experimental import pallas as pl


def kernel(x, edge_index, W1, W2):
    raise NotImplementedError("write your pallas kernel here")


# speedup vs baseline: 15.0424x; 15.0424x over previous
import functools

import jax
import jax.numpy as jnp
from jax import lax
from jax.experimental import pallas as pl
from jax.experimental.pallas import tpu as pltpu
from jax.experimental.pallas import tpu_sc as plsc

N = 10000
E = 320000
D = 128
NHID = 32
F = 16

NC = 2
NS = 16
NW = NC * NS
EP = E // NW
CH = 80
NJ = EP // CH
NP = 10240
RPT = NP // NS


def _make_prop(gather: bool):
    @functools.partial(
        pl.kernel,
        out_type=jax.ShapeDtypeStruct((NC, NP, F), jnp.float32),
        mesh=plsc.VectorSubcoreMesh(core_axis_name="c", subcore_axis_name="s"),
        compiler_params=pltpu.CompilerParams(use_tc_tiling_on_sc=False),
        scratch_types=[
            pltpu.VMEM((NJ, CH), jnp.int32),
            pltpu.VMEM((NJ, CH), jnp.int32),
            pltpu.VMEM((CH, F), jnp.float32),
            pltpu.VMEM_SHARED((NP, F), jnp.float32),
            pltpu.SemaphoreType.DMA,
        ],
    )
    def prop(src_hbm, dst_hbm, table_hbm, zeros_hbm, out_hbm,
             src_v, dst_v, rows_v, acc_sh, gsem):
        c = lax.axis_index("c")
        s = lax.axis_index("s")
        wid = s * NC + c

        pltpu.sync_copy(zeros_hbm.at[pl.ds(s * RPT, RPT)],
                        acc_sh.at[pl.ds(s * RPT, RPT)])
        pltpu.sync_copy(src_hbm.at[wid], src_v)
        pltpu.sync_copy(dst_hbm.at[wid], dst_v)
        if not gather:
            pltpu.sync_copy(table_hbm.at[pl.ds(0, CH)], rows_v)
        plsc.subcore_barrier()

        def body(j, carry):
            if gather:
                pltpu.async_copy(table_hbm.at[src_v.at[j]], rows_v, gsem).wait()
            pltpu.sync_copy(rows_v, acc_sh.at[dst_v.at[j]], add=True)
            return carry

        lax.fori_loop(0, NJ, body, 0)
        plsc.subcore_barrier()

        pltpu.sync_copy(acc_sh.at[pl.ds(s * RPT, RPT)],
                        out_hbm.at[c, pl.ds(s * RPT, RPT)])

    return prop


_prop_gather = _make_prop(gather=True)
_prop_ones = _make_prop(gather=False)


def _prep(x, W1, W2, degP):
    B = 1000

    def body(x_ref, w1_ref, w2_ref, degp_ref, u1_ref, dinvb_ref):
        w12 = jnp.dot(w1_ref[...], w2_ref[...], preferred_element_type=jnp.float32)
        y = jnp.dot(x_ref[...], w12, preferred_element_type=jnp.float32)
        deg = 1.0 + degp_ref[0, :, 0] + degp_ref[1, :, 0]
        dinv = lax.rsqrt(deg)
        u1_ref[...] = y * dinv[:, None]
        dinvb_ref[...] = jnp.broadcast_to(dinv[:, None], (B, F))

    return pl.pallas_call(
        body,
        grid=(N // B,),
        in_specs=[
            pl.BlockSpec((B, D), lambda i: (i, 0)),
            pl.BlockSpec((D, NHID), lambda i: (0, 0)),
            pl.BlockSpec((NHID, F), lambda i: (0, 0)),
            pl.BlockSpec((NC, B, F), lambda i: (0, i, 0)),
        ],
        out_specs=[pl.BlockSpec((B, F), lambda i: (i, 0)),
                   pl.BlockSpec((B, F), lambda i: (i, 0))],
        out_shape=[jax.ShapeDtypeStruct((N, F), jnp.float32),
                   jax.ShapeDtypeStruct((N, F), jnp.float32)],
    )(x, W1, W2, degP)


def _combine(P, uprev, dinvb, square: bool):
    B = 1000

    def body(p_ref, u_ref, d_ref, o_ref):
        sc = d_ref[...]
        if square:
            sc = sc * sc
        o_ref[...] = (p_ref[0] + p_ref[1] + u_ref[...]) * sc

    return pl.pallas_call(
        body,
        grid=(N // B,),
        in_specs=[
            pl.BlockSpec((NC, B, F), lambda i: (0, i, 0)),
            pl.BlockSpec((B, F), lambda i: (i, 0)),
            pl.BlockSpec((B, F), lambda i: (i, 0)),
        ],
        out_specs=pl.BlockSpec((B, F), lambda i: (i, 0)),
        out_shape=jax.ShapeDtypeStruct((N, F), jnp.float32),
    )(P, uprev, dinvb)


def _decoder(z):
    BM = 400

    def body(zm_ref, zall_ref, o_ref):
        a = lax.dot_general(zm_ref[...], zall_ref[...], (((1,), (1,)), ((), ())),
                            preferred_element_type=jnp.float32,
                            precision=lax.Precision.HIGHEST)
        o_ref[...] = jax.nn.sigmoid(a)

    return pl.pallas_call(
        body,
        grid=(N // BM,),
        in_specs=[
            pl.BlockSpec((BM, F), lambda i: (i, 0)),
            pl.BlockSpec((N, F), lambda i: (0, 0)),
        ],
        out_specs=pl.BlockSpec((BM, N), lambda i: (i, 0)),
        out_shape=jax.ShapeDtypeStruct((N, N), jnp.float32),
    )(z, z)


def kernel(x, edge_index, W1, W2):
    ei = edge_index.astype(jnp.int32)
    srcr = ei[0].reshape(NW, NJ, CH)
    dstr = ei[1].reshape(NW, NJ, CH)
    ones_t = jnp.ones((NP, F), jnp.float32)
    zeros_t = jnp.zeros((NP, F), jnp.float32)
    pad = lambda a: jnp.pad(a, ((0, NP - N), (0, 0)))

    degP = _prop_ones(srcr, dstr, ones_t, zeros_t)
    u1, dinvb = _prep(x, W1, W2, degP)
    P = _prop_gather(srcr, dstr, pad(u1), zeros_t)
    u2 = _combine(P, u1, dinvb, square=True)
    Q = _prop_gather(srcr, dstr, pad(u2), zeros_t)
    z = _combine(Q, u2, dinvb, square=False)
    adj = _decoder(z)
    return adj, z


# speedup vs baseline: 30.0396x; 1.9970x over previous
import functools

import jax
import jax.numpy as jnp
from jax import lax
from jax.experimental import pallas as pl
from jax.experimental.pallas import tpu as pltpu
from jax.experimental.pallas import tpu_sc as plsc

N = 10000
E = 320000
D = 128
NHID = 32
F = 16

NC = 2
NS = 16
NW = NC * NS
EP = E // NW
CH = 80
NJ = EP // CH
NBUF = 5
NP = 10240
RPT = NP // NS


def _make_prop(gather: bool):
    @functools.partial(
        pl.kernel,
        out_type=jax.ShapeDtypeStruct((NC, NP, F), jnp.float32),
        mesh=plsc.VectorSubcoreMesh(core_axis_name="c", subcore_axis_name="s"),
        compiler_params=pltpu.CompilerParams(use_tc_tiling_on_sc=False),
        scratch_types=[
            pltpu.VMEM((NJ, CH), jnp.int32),
            pltpu.VMEM((NJ, CH), jnp.int32),
            pltpu.VMEM((NBUF, CH, F), jnp.float32),
            pltpu.VMEM_SHARED((NP, F), jnp.float32),
        ] + [pltpu.SemaphoreType.DMA] * NBUF,
    )
    def prop(src_hbm, dst_hbm, table_hbm, zeros_hbm, out_hbm,
             src_v, dst_v, rows_v, acc_sh, *gsems):
        c = lax.axis_index("c")
        s = lax.axis_index("s")
        wid = s * NC + c

        pltpu.sync_copy(zeros_hbm.at[pl.ds(s * RPT, RPT)],
                        acc_sh.at[pl.ds(s * RPT, RPT)])
        pltpu.sync_copy(src_hbm.at[wid], src_v)
        pltpu.sync_copy(dst_hbm.at[wid], dst_v)
        if not gather:
            for b in range(NBUF):
                pltpu.sync_copy(table_hbm.at[pl.ds(0, CH)], rows_v.at[b])
        plsc.subcore_barrier()

        if gather:
            for b in range(NBUF):
                pltpu.async_copy(table_hbm.at[src_v.at[b]], rows_v.at[b],
                                 gsems[b])

            def body(g, carry):
                for b in range(NBUF):
                    j = g * NBUF + b
                    pltpu.make_async_copy(table_hbm.at[src_v.at[j]],
                                          rows_v.at[b], gsems[b]).wait()
                    pltpu.sync_copy(rows_v.at[b], acc_sh.at[dst_v.at[j]],
                                    add=True)
                    pltpu.async_copy(table_hbm.at[src_v.at[j + NBUF]],
                                     rows_v.at[b], gsems[b])
                return carry

            lax.fori_loop(0, NJ // NBUF - 1, body, 0)
            for b in range(NBUF):
                j = (NJ // NBUF - 1) * NBUF + b
                pltpu.make_async_copy(table_hbm.at[src_v.at[j]],
                                      rows_v.at[b], gsems[b]).wait()
                pltpu.sync_copy(rows_v.at[b], acc_sh.at[dst_v.at[j]],
                                add=True)
        else:
            def body(g, carry):
                for b in range(NBUF):
                    pltpu.sync_copy(rows_v.at[b],
                                    acc_sh.at[dst_v.at[g * NBUF + b]],
                                    add=True)
                return carry

            lax.fori_loop(0, NJ // NBUF, body, 0)
        plsc.subcore_barrier()

        pltpu.sync_copy(acc_sh.at[pl.ds(s * RPT, RPT)],
                        out_hbm.at[c, pl.ds(s * RPT, RPT)])

    return prop


_prop_gather = _make_prop(gather=True)
_prop_ones = _make_prop(gather=False)


def _prep(x, W1, W2, degP):
    B = 1000

    def body(x_ref, w1_ref, w2_ref, degp_ref, u1_ref, dinvb_ref):
        w12 = jnp.dot(w1_ref[...], w2_ref[...], preferred_element_type=jnp.float32)
        y = jnp.dot(x_ref[...], w12, preferred_element_type=jnp.float32)
        deg = 1.0 + degp_ref[0, :, 0] + degp_ref[1, :, 0]
        dinv = lax.rsqrt(deg)
        u1_ref[...] = y * dinv[:, None]
        dinvb_ref[...] = jnp.broadcast_to(dinv[:, None], (B, F))

    return pl.pallas_call(
        body,
        grid=(N // B,),
        in_specs=[
            pl.BlockSpec((B, D), lambda i: (i, 0)),
            pl.BlockSpec((D, NHID), lambda i: (0, 0)),
            pl.BlockSpec((NHID, F), lambda i: (0, 0)),
            pl.BlockSpec((NC, B, F), lambda i: (0, i, 0)),
        ],
        out_specs=[pl.BlockSpec((B, F), lambda i: (i, 0)),
                   pl.BlockSpec((B, F), lambda i: (i, 0))],
        out_shape=[jax.ShapeDtypeStruct((N, F), jnp.float32),
                   jax.ShapeDtypeStruct((N, F), jnp.float32)],
    )(x, W1, W2, degP)


def _combine(P, uprev, dinvb, square: bool):
    B = 1000

    def body(p_ref, u_ref, d_ref, o_ref):
        sc = d_ref[...]
        if square:
            sc = sc * sc
        o_ref[...] = (p_ref[0] + p_ref[1] + u_ref[...]) * sc

    return pl.pallas_call(
        body,
        grid=(N // B,),
        in_specs=[
            pl.BlockSpec((NC, B, F), lambda i: (0, i, 0)),
            pl.BlockSpec((B, F), lambda i: (i, 0)),
            pl.BlockSpec((B, F), lambda i: (i, 0)),
        ],
        out_specs=pl.BlockSpec((B, F), lambda i: (i, 0)),
        out_shape=jax.ShapeDtypeStruct((N, F), jnp.float32),
    )(P, uprev, dinvb)


def _decoder(z):
    BM = 400

    def body(zm_ref, zall_ref, o_ref):
        a = lax.dot_general(zm_ref[...], zall_ref[...], (((1,), (1,)), ((), ())),
                            preferred_element_type=jnp.float32,
                            precision=lax.Precision.DEFAULT)
        o_ref[...] = jax.nn.sigmoid(a)

    return pl.pallas_call(
        body,
        grid=(N // BM,),
        in_specs=[
            pl.BlockSpec((BM, F), lambda i: (i, 0)),
            pl.BlockSpec((N, F), lambda i: (0, 0)),
        ],
        out_specs=pl.BlockSpec((BM, N), lambda i: (i, 0)),
        out_shape=jax.ShapeDtypeStruct((N, N), jnp.float32),
    )(z, z)


def kernel(x, edge_index, W1, W2):
    ei = edge_index.astype(jnp.int32)
    srcr = ei[0].reshape(NW, NJ, CH)
    dstr = ei[1].reshape(NW, NJ, CH)
    ones_t = jnp.ones((NP, F), jnp.float32)
    zeros_t = jnp.zeros((NP, F), jnp.float32)
    pad = lambda a: jnp.pad(a, ((0, NP - N), (0, 0)))

    degP = _prop_ones(srcr, dstr, ones_t, zeros_t)
    u1, dinvb = _prep(x, W1, W2, degP)
    P = _prop_gather(srcr, dstr, pad(u1), zeros_t)
    u2 = _combine(P, u1, dinvb, square=True)
    Q = _prop_gather(srcr, dstr, pad(u2), zeros_t)
    z = _combine(Q, u2, dinvb, square=False)
    adj = _decoder(z)
    return adj, z


# speedup vs baseline: 31.2465x; 1.0402x over previous
import functools

import jax
import jax.numpy as jnp
from jax import lax
from jax.experimental import pallas as pl
from jax.experimental.pallas import tpu as pltpu
from jax.experimental.pallas import tpu_sc as plsc

N = 10000
E = 320000
D = 128
NHID = 32
F = 16

NC = 2
NS = 16
NW = NC * NS
EP = E // NW
CH = 80
NJ = EP // CH
NBUF = 5
NP = 10240
RPT = NP // NS


def _make_prop(gather: bool):
    @functools.partial(
        pl.kernel,
        out_type=jax.ShapeDtypeStruct((NC, NP, F), jnp.float32),
        mesh=plsc.VectorSubcoreMesh(core_axis_name="c", subcore_axis_name="s"),
        compiler_params=pltpu.CompilerParams(use_tc_tiling_on_sc=False),
        scratch_types=[
            pltpu.VMEM((NJ, CH), jnp.int32),
            pltpu.VMEM((NJ, CH), jnp.int32),
            pltpu.VMEM((NBUF, CH, F), jnp.float32),
            pltpu.VMEM_SHARED((NP, F), jnp.float32),
        ] + [pltpu.SemaphoreType.DMA] * NBUF,
    )
    def prop(src_hbm, dst_hbm, table_hbm, zeros_hbm, out_hbm,
             src_v, dst_v, rows_v, acc_sh, *gsems):
        c = lax.axis_index("c")
        s = lax.axis_index("s")
        wid = s * NC + c

        pltpu.sync_copy(zeros_hbm.at[pl.ds(s * RPT, RPT)],
                        acc_sh.at[pl.ds(s * RPT, RPT)])
        pltpu.sync_copy(src_hbm.at[wid], src_v)
        pltpu.sync_copy(dst_hbm.at[wid], dst_v)
        if not gather:
            for b in range(NBUF):
                pltpu.sync_copy(table_hbm.at[pl.ds(0, CH)], rows_v.at[b])
        plsc.subcore_barrier()

        if gather:
            for b in range(NBUF):
                pltpu.async_copy(table_hbm.at[src_v.at[b]], rows_v.at[b],
                                 gsems[b])

            def body(g, carry):
                for b in range(NBUF):
                    j = g * NBUF + b
                    pltpu.make_async_copy(table_hbm.at[src_v.at[j]],
                                          rows_v.at[b], gsems[b]).wait()
                    pltpu.sync_copy(rows_v.at[b], acc_sh.at[dst_v.at[j]],
                                    add=True)
                    pltpu.async_copy(table_hbm.at[src_v.at[j + NBUF]],
                                     rows_v.at[b], gsems[b])
                return carry

            lax.fori_loop(0, NJ // NBUF - 1, body, 0)
            for b in range(NBUF):
                j = (NJ // NBUF - 1) * NBUF + b
                pltpu.make_async_copy(table_hbm.at[src_v.at[j]],
                                      rows_v.at[b], gsems[b]).wait()
                pltpu.sync_copy(rows_v.at[b], acc_sh.at[dst_v.at[j]],
                                add=True)
        else:
            def body(g, carry):
                for b in range(NBUF):
                    pltpu.sync_copy(rows_v.at[b],
                                    acc_sh.at[dst_v.at[g * NBUF + b]],
                                    add=True)
                return carry

            lax.fori_loop(0, NJ // NBUF, body, 0)
        plsc.subcore_barrier()

        pltpu.sync_copy(acc_sh.at[pl.ds(s * RPT, RPT)],
                        out_hbm.at[c, pl.ds(s * RPT, RPT)])

    return prop


_prop_gather = _make_prop(gather=True)
_prop_ones = _make_prop(gather=False)


def _prep(x, W1, W2, degP):
    B = 1000

    def body(x_ref, w1_ref, w2_ref, degp_ref, u1_ref, dinvb_ref):
        w12 = jnp.dot(w1_ref[...], w2_ref[...], preferred_element_type=jnp.float32)
        y = jnp.dot(x_ref[...], w12, preferred_element_type=jnp.float32)
        deg = 1.0 + degp_ref[0, :, 0] + degp_ref[1, :, 0]
        dinv = lax.rsqrt(deg)
        u1_ref[...] = y * dinv[:, None]
        dinvb_ref[...] = jnp.broadcast_to(dinv[:, None], (B, F))

    return pl.pallas_call(
        body,
        grid=(N // B,),
        in_specs=[
            pl.BlockSpec((B, D), lambda i: (i, 0)),
            pl.BlockSpec((D, NHID), lambda i: (0, 0)),
            pl.BlockSpec((NHID, F), lambda i: (0, 0)),
            pl.BlockSpec((NC, B, F), lambda i: (0, i, 0)),
        ],
        out_specs=[pl.BlockSpec((B, F), lambda i: (i, 0)),
                   pl.BlockSpec((B, F), lambda i: (i, 0))],
        out_shape=[jax.ShapeDtypeStruct((N, F), jnp.float32),
                   jax.ShapeDtypeStruct((N, F), jnp.float32)],
    )(x, W1, W2, degP)


def _combine(P, uprev, dinvb, square: bool):
    B = 1000

    def body(p_ref, u_ref, d_ref, o_ref):
        sc = d_ref[...]
        if square:
            sc = sc * sc
        o_ref[...] = (p_ref[0] + p_ref[1] + u_ref[...]) * sc

    return pl.pallas_call(
        body,
        grid=(N // B,),
        in_specs=[
            pl.BlockSpec((NC, B, F), lambda i: (0, i, 0)),
            pl.BlockSpec((B, F), lambda i: (i, 0)),
            pl.BlockSpec((B, F), lambda i: (i, 0)),
        ],
        out_specs=pl.BlockSpec((B, F), lambda i: (i, 0)),
        out_shape=jax.ShapeDtypeStruct((N, F), jnp.float32),
    )(P, uprev, dinvb)


def _decoder(z):
    BM = 400

    def body(zm_ref, zall_ref, o_ref):
        a = lax.dot_general(zm_ref[...], zall_ref[...], (((1,), (1,)), ((), ())),
                            preferred_element_type=jnp.float32,
                            precision=lax.Precision.DEFAULT)
        o_ref[...] = 0.5 + 0.5 * lax.tanh(0.5 * a)

    return pl.pallas_call(
        body,
        grid=(N // BM,),
        in_specs=[
            pl.BlockSpec((BM, F), lambda i: (i, 0)),
            pl.BlockSpec((N, F), lambda i: (0, 0)),
        ],
        out_specs=pl.BlockSpec((BM, N), lambda i: (i, 0)),
        out_shape=jax.ShapeDtypeStruct((N, N), jnp.float32),
    )(z, z)


def kernel(x, edge_index, W1, W2):
    ei = edge_index.astype(jnp.int32)
    srcr = ei[0].reshape(NW, NJ, CH)
    dstr = ei[1].reshape(NW, NJ, CH)
    ones_t = jnp.ones((NP, F), jnp.float32)
    zeros_t = jnp.zeros((NP, F), jnp.float32)
    pad = lambda a: jnp.pad(a, ((0, NP - N), (0, 0)))

    degP = _prop_ones(srcr, dstr, ones_t, zeros_t)
    u1, dinvb = _prep(x, W1, W2, degP)
    P = _prop_gather(srcr, dstr, pad(u1), zeros_t)
    u2 = _combine(P, u1, dinvb, square=True)
    Q = _prop_gather(srcr, dstr, pad(u2), zeros_t)
    z = _combine(Q, u2, dinvb, square=False)
    adj = _decoder(z)
    return adj, z


# speedup vs baseline: 33.8438x; 1.0831x over previous
import functools

import jax
import jax.numpy as jnp
from jax import lax
from jax.experimental import pallas as pl
from jax.experimental.pallas import tpu as pltpu
from jax.experimental.pallas import tpu_sc as plsc

N = 10000
E = 320000
D = 128
NHID = 32
F = 16

NC = 2
NS = 16
NW = NC * NS
EP = E // NW
CH = 80
NJ = EP // CH
NBUF = 5
NP = 10240
RPT = NP // NS
L = 16


@functools.partial(
    pl.kernel,
    out_type=jax.ShapeDtypeStruct((NW, NP), jnp.float32),
    mesh=plsc.VectorSubcoreMesh(core_axis_name="c", subcore_axis_name="s"),
    compiler_params=pltpu.CompilerParams(use_tc_tiling_on_sc=False,
                                         needs_layout_passes=False),
    scratch_types=[
        pltpu.VMEM((EP,), jnp.int32),
        pltpu.VMEM((NP,), jnp.float32),
    ],
)
def _deg(dst_hbm, zeros_hbm, out_hbm, dst_v, deg_v):
    c = lax.axis_index("c")
    s = lax.axis_index("s")
    wid = s * NC + c

    pltpu.sync_copy(zeros_hbm.at[pl.ds(0, NP)], deg_v)
    pltpu.sync_copy(dst_hbm.at[wid], dst_v)
    ones = jnp.ones((L,), jnp.float32)

    def body(i, carry):
        idx = dst_v[pl.ds(i * L, L)]
        plsc.addupdate_scatter(deg_v, [idx], ones)
        return carry

    lax.fori_loop(0, EP // L, body, 0)
    pltpu.sync_copy(deg_v, out_hbm.at[wid])


@functools.partial(
    pl.kernel,
    out_type=jax.ShapeDtypeStruct((NC, NP, F), jnp.float32),
    mesh=plsc.VectorSubcoreMesh(core_axis_name="c", subcore_axis_name="s"),
    compiler_params=pltpu.CompilerParams(use_tc_tiling_on_sc=False),
    scratch_types=[
        pltpu.VMEM((NJ, CH), jnp.int32),
        pltpu.VMEM((NJ, CH), jnp.int32),
        pltpu.VMEM((NBUF, CH, F), jnp.float32),
        pltpu.VMEM_SHARED((NP, F), jnp.float32),
    ] + [pltpu.SemaphoreType.DMA] * NBUF,
)
def _prop(src_hbm, dst_hbm, table_hbm, zeros_hbm, out_hbm,
          src_v, dst_v, rows_v, acc_sh, *gsems):
    c = lax.axis_index("c")
    s = lax.axis_index("s")
    wid = s * NC + c

    pltpu.sync_copy(zeros_hbm.at[pl.ds(s * RPT, RPT)],
                    acc_sh.at[pl.ds(s * RPT, RPT)])
    pltpu.sync_copy(src_hbm.at[wid], src_v)
    pltpu.sync_copy(dst_hbm.at[wid], dst_v)
    plsc.subcore_barrier()

    for b in range(NBUF):
        pltpu.async_copy(table_hbm.at[src_v.at[b]], rows_v.at[b], gsems[b])

    def body(g, carry):
        for b in range(NBUF):
            j = g * NBUF + b
            pltpu.make_async_copy(table_hbm.at[src_v.at[j]],
                                  rows_v.at[b], gsems[b]).wait()
            pltpu.sync_copy(rows_v.at[b], acc_sh.at[dst_v.at[j]], add=True)
            pltpu.async_copy(table_hbm.at[src_v.at[j + NBUF]],
                             rows_v.at[b], gsems[b])
        return carry

    lax.fori_loop(0, NJ // NBUF - 1, body, 0)
    for b in range(NBUF):
        j = (NJ // NBUF - 1) * NBUF + b
        pltpu.make_async_copy(table_hbm.at[src_v.at[j]],
                              rows_v.at[b], gsems[b]).wait()
        pltpu.sync_copy(rows_v.at[b], acc_sh.at[dst_v.at[j]], add=True)
    plsc.subcore_barrier()

    pltpu.sync_copy(acc_sh.at[pl.ds(s * RPT, RPT)],
                    out_hbm.at[c, pl.ds(s * RPT, RPT)])


def _prep(xp, W1, W2, degW):
    B = 1024

    def body(x_ref, w1_ref, w2_ref, degp_ref, u1_ref, dinvb_ref):
        w12 = jnp.dot(w1_ref[...], w2_ref[...], preferred_element_type=jnp.float32)
        y = jnp.dot(x_ref[...], w12, preferred_element_type=jnp.float32)
        deg = 1.0 + jnp.sum(degp_ref[...], axis=0)
        dinv = lax.rsqrt(deg)
        u1_ref[...] = y * dinv[:, None]
        dinvb_ref[...] = jnp.broadcast_to(dinv[:, None], (B, F))

    return pl.pallas_call(
        body,
        grid=(NP // B,),
        in_specs=[
            pl.BlockSpec((B, D), lambda i: (i, 0)),
            pl.BlockSpec((D, NHID), lambda i: (0, 0)),
            pl.BlockSpec((NHID, F), lambda i: (0, 0)),
            pl.BlockSpec((NW, B), lambda i: (0, i)),
        ],
        out_specs=[pl.BlockSpec((B, F), lambda i: (i, 0)),
                   pl.BlockSpec((B, F), lambda i: (i, 0))],
        out_shape=[jax.ShapeDtypeStruct((NP, F), jnp.float32),
                   jax.ShapeDtypeStruct((NP, F), jnp.float32)],
    )(xp, W1, W2, degW)


def _combine(P, uprev, dinvb, square: bool):
    B = 1024

    def body(p_ref, u_ref, d_ref, o_ref):
        sc = d_ref[...]
        if square:
            sc = sc * sc
        o_ref[...] = (p_ref[0] + p_ref[1] + u_ref[...]) * sc

    return pl.pallas_call(
        body,
        grid=(NP // B,),
        in_specs=[
            pl.BlockSpec((NC, B, F), lambda i: (0, i, 0)),
            pl.BlockSpec((B, F), lambda i: (i, 0)),
            pl.BlockSpec((B, F), lambda i: (i, 0)),
        ],
        out_specs=pl.BlockSpec((B, F), lambda i: (i, 0)),
        out_shape=jax.ShapeDtypeStruct((NP, F), jnp.float32),
    )(P, uprev, dinvb)


def _decoder(z):
    BM = 400

    def body(zm_ref, zall_ref, o_ref):
        a = lax.dot_general(zm_ref[...], zall_ref[...], (((1,), (1,)), ((), ())),
                            preferred_element_type=jnp.float32,
                            precision=lax.Precision.DEFAULT)
        o_ref[...] = 0.5 + 0.5 * lax.tanh(0.5 * a)

    return pl.pallas_call(
        body,
        grid=(N // BM,),
        in_specs=[
            pl.BlockSpec((BM, F), lambda i: (i, 0)),
            pl.BlockSpec((N, F), lambda i: (0, 0)),
        ],
        out_specs=pl.BlockSpec((BM, N), lambda i: (i, 0)),
        out_shape=jax.ShapeDtypeStruct((N, N), jnp.float32),
    )(z, z)


def kernel(x, edge_index, W1, W2):
    ei = edge_index.astype(jnp.int32)
    srcr = ei[0].reshape(NW, NJ, CH)
    dstr = ei[1].reshape(NW, NJ, CH)
    dstw = ei[1].reshape(NW, EP)
    zeros_t = jnp.zeros((NP, F), jnp.float32)
    zeros_n = jnp.zeros((NP,), jnp.float32)
    xp = jnp.pad(x, ((0, NP - N), (0, 0)))

    degW = _deg(dstw, zeros_n)
    u1, dinvb = _prep(xp, W1, W2, degW)
    P = _prop(srcr, dstr, u1, zeros_t)
    u2 = _combine(P, u1, dinvb, square=True)
    Q = _prop(srcr, dstr, u2, zeros_t)
    z = _combine(Q, u2, dinvb, square=False)
    adj = _decoder(z)
    return adj, z[:N]


# speedup vs baseline: 34.6372x; 1.0234x over previous
import functools

import jax
import jax.numpy as jnp
from jax import lax
from jax.experimental import pallas as pl
from jax.experimental.pallas import tpu as pltpu
from jax.experimental.pallas import tpu_sc as plsc

N = 10000
E = 320000
D = 128
NHID = 32
F = 16

NC = 2
NS = 16
NW = NC * NS
EP = E // NW
CH = 80
NJ = EP // CH
NB = 10
PD = 5
NP = 10240
RPT = NP // NS
L = 16


@functools.partial(
    pl.kernel,
    out_type=jax.ShapeDtypeStruct((NW, NP), jnp.float32),
    mesh=plsc.VectorSubcoreMesh(core_axis_name="c", subcore_axis_name="s"),
    compiler_params=pltpu.CompilerParams(use_tc_tiling_on_sc=False,
                                         needs_layout_passes=False),
    scratch_types=[
        pltpu.VMEM((EP,), jnp.int32),
        pltpu.VMEM((NP,), jnp.float32),
    ],
)
def _deg(dst_hbm, zeros_hbm, out_hbm, dst_v, deg_v):
    c = lax.axis_index("c")
    s = lax.axis_index("s")
    wid = s * NC + c

    pltpu.sync_copy(zeros_hbm.at[pl.ds(0, NP)], deg_v)
    pltpu.sync_copy(dst_hbm.at[wid], dst_v)
    ones = jnp.ones((L,), jnp.float32)

    def body(i, carry):
        idx = dst_v[pl.ds(i * L, L)]
        plsc.addupdate_scatter(deg_v, [idx], ones)
        return carry

    lax.fori_loop(0, EP // L, body, 0)
    pltpu.sync_copy(deg_v, out_hbm.at[wid])


@functools.partial(
    pl.kernel,
    out_type=jax.ShapeDtypeStruct((NC, NP, F), jnp.float32),
    mesh=plsc.VectorSubcoreMesh(core_axis_name="c", subcore_axis_name="s"),
    compiler_params=pltpu.CompilerParams(use_tc_tiling_on_sc=False),
    scratch_types=[
        pltpu.VMEM((NJ, CH), jnp.int32),
        pltpu.VMEM((NJ, CH), jnp.int32),
        pltpu.VMEM((NB, CH, F), jnp.float32),
        pltpu.VMEM_SHARED((NP, F), jnp.float32),
    ] + [pltpu.SemaphoreType.DMA] * (2 * NB),
)
def _prop(src_hbm, dst_hbm, table_hbm, zeros_hbm, out_hbm,
          src_v, dst_v, rows_v, acc_sh, *sems):
    gsems, ssems = sems[:NB], sems[NB:]
    c = lax.axis_index("c")
    s = lax.axis_index("s")
    wid = s * NC + c

    def gather(j, b):
        pltpu.async_copy(table_hbm.at[src_v.at[j]], rows_v.at[b], gsems[b])

    def gather_wait(j, b):
        pltpu.make_async_copy(table_hbm.at[src_v.at[j]], rows_v.at[b],
                              gsems[b]).wait()

    def scatter(j, b):
        pltpu.async_copy(rows_v.at[b], acc_sh.at[dst_v.at[j]], ssems[b],
                         add=True)

    def scatter_wait(j, b):
        pltpu.make_async_copy(rows_v.at[b], acc_sh.at[dst_v.at[j]],
                              ssems[b]).wait()

    pltpu.sync_copy(zeros_hbm.at[pl.ds(s * RPT, RPT)],
                    acc_sh.at[pl.ds(s * RPT, RPT)])
    pltpu.sync_copy(src_hbm.at[wid], src_v)
    pltpu.sync_copy(dst_hbm.at[wid], dst_v)
    plsc.subcore_barrier()

    for j in range(PD):
        gather(j, j)
    for j in range(NB):
        gather_wait(j, j)
        scatter(j, j)
        if j + PD < NB:
            gather(j + PD, j + PD)
        else:
            scatter_wait(j + PD - NB, (j + PD) % NB)
            gather(j + PD, (j + PD) % NB)

    def body(g, carry):
        for b in range(NB):
            j = g * NB + b
            gather_wait(j, b)
            scatter(j, b)
            bb = (b + PD) % NB
            scatter_wait(j + PD - NB, bb)
            gather(j + PD, bb)
        return carry

    lax.fori_loop(1, (NJ - PD) // NB, body, 0)

    TAIL0 = NB * ((NJ - PD) // NB)
    for j in range(TAIL0, NJ):
        b = j % NB
        gather_wait(j, b)
        scatter(j, b)
        if j + PD < NJ:
            bb = (j + PD) % NB
            scatter_wait(j + PD - NB, bb)
            gather(j + PD, bb)
    for j in range(NJ - NB, NJ):
        scatter_wait(j, j % NB)
    plsc.subcore_barrier()

    pltpu.sync_copy(acc_sh.at[pl.ds(s * RPT, RPT)],
                    out_hbm.at[c, pl.ds(s * RPT, RPT)])


def _prep(xp, W1, W2, degW):
    B = 1024

    def body(x_ref, w1_ref, w2_ref, degp_ref, u1_ref, dinvb_ref):
        w12 = jnp.dot(w1_ref[...], w2_ref[...], preferred_element_type=jnp.float32)
        y = jnp.dot(x_ref[...], w12, preferred_element_type=jnp.float32)
        deg = 1.0 + jnp.sum(degp_ref[...], axis=0)
        dinv = lax.rsqrt(deg)
        u1_ref[...] = y * dinv[:, None]
        dinvb_ref[...] = jnp.broadcast_to(dinv[:, None], (B, F))

    return pl.pallas_call(
        body,
        grid=(NP // B,),
        in_specs=[
            pl.BlockSpec((B, D), lambda i: (i, 0)),
            pl.BlockSpec((D, NHID), lambda i: (0, 0)),
            pl.BlockSpec((NHID, F), lambda i: (0, 0)),
            pl.BlockSpec((NW, B), lambda i: (0, i)),
        ],
        out_specs=[pl.BlockSpec((B, F), lambda i: (i, 0)),
                   pl.BlockSpec((B, F), lambda i: (i, 0))],
        out_shape=[jax.ShapeDtypeStruct((NP, F), jnp.float32),
                   jax.ShapeDtypeStruct((NP, F), jnp.float32)],
    )(xp, W1, W2, degW)


def _combine(P, uprev, dinvb, square: bool):
    B = 1024

    def body(p_ref, u_ref, d_ref, o_ref):
        sc = d_ref[...]
        if square:
            sc = sc * sc
        o_ref[...] = (p_ref[0] + p_ref[1] + u_ref[...]) * sc

    return pl.pallas_call(
        body,
        grid=(NP // B,),
        in_specs=[
            pl.BlockSpec((NC, B, F), lambda i: (0, i, 0)),
            pl.BlockSpec((B, F), lambda i: (i, 0)),
            pl.BlockSpec((B, F), lambda i: (i, 0)),
        ],
        out_specs=pl.BlockSpec((B, F), lambda i: (i, 0)),
        out_shape=jax.ShapeDtypeStruct((NP, F), jnp.float32),
    )(P, uprev, dinvb)


def _decoder(Q, u2, dinvb):
    BM = 400

    def body(q_ref, u_ref, d_ref, o_ref, z_ref, zs_ref):
        i = pl.program_id(0)

        @pl.when(i == 0)
        def _():
            zfull = (q_ref[0] + q_ref[1] + u_ref[...]) * d_ref[...]
            zs_ref[...] = zfull
            z_ref[...] = zfull

        zm = zs_ref[pl.ds(i * BM, BM), :]
        a = lax.dot_general(zm, zs_ref[:N, :], (((1,), (1,)), ((), ())),
                            preferred_element_type=jnp.float32,
                            precision=lax.Precision.DEFAULT)
        o_ref[...] = 0.5 + 0.5 * lax.tanh(0.5 * a)

    return pl.pallas_call(
        body,
        grid=(N // BM,),
        in_specs=[
            pl.BlockSpec((NC, NP, F), lambda i: (0, 0, 0)),
            pl.BlockSpec((NP, F), lambda i: (0, 0)),
            pl.BlockSpec((NP, F), lambda i: (0, 0)),
        ],
        out_specs=[pl.BlockSpec((BM, N), lambda i: (i, 0)),
                   pl.BlockSpec((NP, F), lambda i: (0, 0))],
        out_shape=[jax.ShapeDtypeStruct((N, N), jnp.float32),
                   jax.ShapeDtypeStruct((NP, F), jnp.float32)],
        scratch_shapes=[pltpu.VMEM((NP, F), jnp.float32)],
        compiler_params=pltpu.CompilerParams(
            vmem_limit_bytes=100 * 1024 * 1024),
    )(Q, u2, dinvb)


def kernel(x, edge_index, W1, W2):
    ei = edge_index.astype(jnp.int32)
    srcr = ei[0].reshape(NW, NJ, CH)
    dstr = ei[1].reshape(NW, NJ, CH)
    dstw = ei[1].reshape(NW, EP)
    zeros_t = jnp.zeros((NP, F), jnp.float32)
    zeros_n = jnp.zeros((NP,), jnp.float32)
    xp = jnp.pad(x, ((0, NP - N), (0, 0)))

    degW = _deg(dstw, zeros_n)
    u1, dinvb = _prep(xp, W1, W2, degW)
    P = _prop(srcr, dstr, u1, zeros_t)
    u2 = _combine(P, u1, dinvb, square=True)
    Q = _prop(srcr, dstr, u2, zeros_t)
    adj, z = _decoder(Q, u2, dinvb)
    return adj, z[:N]


# speedup vs baseline: 36.4985x; 1.0537x over previous
import functools

import jax
import jax.numpy as jnp
from jax import lax
from jax.experimental import pallas as pl
from jax.experimental.pallas import tpu as pltpu
from jax.experimental.pallas import tpu_sc as plsc

N = 10000
E = 320000
D = 128
NHID = 32
F = 16

NC = 2
NS = 16
NW = NC * NS
EP = E // NW
CH = 80
NJ = EP // CH
NB = 10
PD = 5
NP = 10240
RPT = NP // NS
L = 16


@functools.partial(
    pl.kernel,
    out_type=jax.ShapeDtypeStruct((NW, NP), jnp.float32),
    mesh=plsc.VectorSubcoreMesh(core_axis_name="c", subcore_axis_name="s"),
    compiler_params=pltpu.CompilerParams(use_tc_tiling_on_sc=False,
                                         needs_layout_passes=False),
    scratch_types=[
        pltpu.VMEM((EP,), jnp.int32),
        pltpu.VMEM((NP,), jnp.float32),
    ],
)
def _deg(dst_hbm, zeros_hbm, out_hbm, dst_v, deg_v):
    c = lax.axis_index("c")
    s = lax.axis_index("s")
    wid = s * NC + c

    pltpu.sync_copy(zeros_hbm.at[pl.ds(0, NP)], deg_v)
    pltpu.sync_copy(dst_hbm.at[wid], dst_v)
    ones = jnp.ones((L,), jnp.float32)

    def body(i, carry):
        idx = dst_v[pl.ds(i * L, L)]
        plsc.addupdate_scatter(deg_v, [idx], ones)
        return carry

    lax.fori_loop(0, EP // L, body, 0)
    pltpu.sync_copy(deg_v, out_hbm.at[wid])


@functools.partial(
    pl.kernel,
    out_type=jax.ShapeDtypeStruct((NC, NP, F), jnp.float32),
    mesh=plsc.VectorSubcoreMesh(core_axis_name="c", subcore_axis_name="s"),
    compiler_params=pltpu.CompilerParams(use_tc_tiling_on_sc=False),
    scratch_types=[
        pltpu.VMEM((NJ, CH), jnp.int32),
        pltpu.VMEM((NJ, CH), jnp.int32),
        pltpu.VMEM((NB, CH, F), jnp.float32),
        pltpu.VMEM_SHARED((NP, F), jnp.float32),
        pltpu.VMEM_SHARED((NP, F), jnp.float32),
    ] + [pltpu.SemaphoreType.DMA] * (2 * NB),
)
def _prop(src_hbm, dst_hbm, table_hbm, zeros_hbm, out_hbm,
          src_v, dst_v, rows_v, acc_sh, table_sh, *sems):
    gsems, ssems = sems[:NB], sems[NB:]
    c = lax.axis_index("c")
    s = lax.axis_index("s")
    wid = s * NC + c

    def gather(j, b):
        pltpu.async_copy(table_sh.at[src_v.at[j]], rows_v.at[b], gsems[b])

    def gather_wait(j, b):
        pltpu.make_async_copy(table_sh.at[src_v.at[j]], rows_v.at[b],
                              gsems[b]).wait()

    def scatter(j, b):
        pltpu.async_copy(rows_v.at[b], acc_sh.at[dst_v.at[j]], ssems[b],
                         add=True)

    def scatter_wait(j, b):
        pltpu.make_async_copy(rows_v.at[b], acc_sh.at[dst_v.at[j]],
                              ssems[b]).wait()

    pltpu.sync_copy(zeros_hbm.at[pl.ds(s * RPT, RPT)],
                    acc_sh.at[pl.ds(s * RPT, RPT)])
    pltpu.sync_copy(table_hbm.at[pl.ds(s * RPT, RPT)],
                    table_sh.at[pl.ds(s * RPT, RPT)])
    pltpu.sync_copy(src_hbm.at[wid], src_v)
    pltpu.sync_copy(dst_hbm.at[wid], dst_v)
    plsc.subcore_barrier()

    for j in range(PD):
        gather(j, j)
    for j in range(NB):
        gather_wait(j, j)
        scatter(j, j)
        if j + PD < NB:
            gather(j + PD, j + PD)
        else:
            scatter_wait(j + PD - NB, (j + PD) % NB)
            gather(j + PD, (j + PD) % NB)

    def body(g, carry):
        for b in range(NB):
            j = g * NB + b
            gather_wait(j, b)
            scatter(j, b)
            bb = (b + PD) % NB
            scatter_wait(j + PD - NB, bb)
            gather(j + PD, bb)
        return carry

    lax.fori_loop(1, (NJ - PD) // NB, body, 0)

    TAIL0 = NB * ((NJ - PD) // NB)
    for j in range(TAIL0, NJ):
        b = j % NB
        gather_wait(j, b)
        scatter(j, b)
        if j + PD < NJ:
            bb = (j + PD) % NB
            scatter_wait(j + PD - NB, bb)
            gather(j + PD, bb)
    for j in range(NJ - NB, NJ):
        scatter_wait(j, j % NB)
    plsc.subcore_barrier()

    pltpu.sync_copy(acc_sh.at[pl.ds(s * RPT, RPT)],
                    out_hbm.at[c, pl.ds(s * RPT, RPT)])


def _prep(xp, W1, W2, degW):
    B = 1024

    def body(x_ref, w1_ref, w2_ref, degp_ref, u1_ref, dinvb_ref):
        w12 = jnp.dot(w1_ref[...], w2_ref[...], preferred_element_type=jnp.float32)
        y = jnp.dot(x_ref[...], w12, preferred_element_type=jnp.float32)
        deg = 1.0 + jnp.sum(degp_ref[...], axis=0)
        dinv = lax.rsqrt(deg)
        u1_ref[...] = y * dinv[:, None]
        dinvb_ref[...] = jnp.broadcast_to(dinv[:, None], (B, F))

    return pl.pallas_call(
        body,
        grid=(NP // B,),
        in_specs=[
            pl.BlockSpec((B, D), lambda i: (i, 0)),
            pl.BlockSpec((D, NHID), lambda i: (0, 0)),
            pl.BlockSpec((NHID, F), lambda i: (0, 0)),
            pl.BlockSpec((NW, B), lambda i: (0, i)),
        ],
        out_specs=[pl.BlockSpec((B, F), lambda i: (i, 0)),
                   pl.BlockSpec((B, F), lambda i: (i, 0))],
        out_shape=[jax.ShapeDtypeStruct((NP, F), jnp.float32),
                   jax.ShapeDtypeStruct((NP, F), jnp.float32)],
    )(xp, W1, W2, degW)


def _combine(P, uprev, dinvb, square: bool):
    B = 1024

    def body(p_ref, u_ref, d_ref, o_ref):
        sc = d_ref[...]
        if square:
            sc = sc * sc
        o_ref[...] = (p_ref[0] + p_ref[1] + u_ref[...]) * sc

    return pl.pallas_call(
        body,
        grid=(NP // B,),
        in_specs=[
            pl.BlockSpec((NC, B, F), lambda i: (0, i, 0)),
            pl.BlockSpec((B, F), lambda i: (i, 0)),
            pl.BlockSpec((B, F), lambda i: (i, 0)),
        ],
        out_specs=pl.BlockSpec((B, F), lambda i: (i, 0)),
        out_shape=jax.ShapeDtypeStruct((NP, F), jnp.float32),
    )(P, uprev, dinvb)


def _decoder(Q, u2, dinvb):
    BM = 400

    def body(q_ref, u_ref, d_ref, o_ref, z_ref, zs_ref):
        i = pl.program_id(0)

        @pl.when(i == 0)
        def _():
            zfull = (q_ref[0] + q_ref[1] + u_ref[...]) * d_ref[...]
            zs_ref[...] = zfull
            z_ref[...] = zfull

        zm = zs_ref[pl.ds(i * BM, BM), :]
        a = lax.dot_general(zm, zs_ref[:N, :], (((1,), (1,)), ((), ())),
                            preferred_element_type=jnp.float32,
                            precision=lax.Precision.DEFAULT)
        o_ref[...] = 0.5 + 0.5 * lax.tanh(0.5 * a)

    return pl.pallas_call(
        body,
        grid=(N // BM,),
        in_specs=[
            pl.BlockSpec((NC, NP, F), lambda i: (0, 0, 0)),
            pl.BlockSpec((NP, F), lambda i: (0, 0)),
            pl.BlockSpec((NP, F), lambda i: (0, 0)),
        ],
        out_specs=[pl.BlockSpec((BM, N), lambda i: (i, 0)),
                   pl.BlockSpec((NP, F), lambda i: (0, 0))],
        out_shape=[jax.ShapeDtypeStruct((N, N), jnp.float32),
                   jax.ShapeDtypeStruct((NP, F), jnp.float32)],
        scratch_shapes=[pltpu.VMEM((NP, F), jnp.float32)],
        compiler_params=pltpu.CompilerParams(
            vmem_limit_bytes=100 * 1024 * 1024),
    )(Q, u2, dinvb)


def kernel(x, edge_index, W1, W2):
    ei = edge_index.astype(jnp.int32)
    srcr = ei[0].reshape(NW, NJ, CH)
    dstr = ei[1].reshape(NW, NJ, CH)
    dstw = ei[1].reshape(NW, EP)
    zeros_t = jnp.zeros((NP, F), jnp.float32)
    zeros_n = jnp.zeros((NP,), jnp.float32)
    xp = jnp.pad(x, ((0, NP - N), (0, 0)))

    degW = _deg(dstw, zeros_n)
    u1, dinvb = _prep(xp, W1, W2, degW)
    P = _prop(srcr, dstr, u1, zeros_t)
    u2 = _combine(P, u1, dinvb, square=True)
    Q = _prop(srcr, dstr, u2, zeros_t)
    adj, z = _decoder(Q, u2, dinvb)
    return adj, z[:N]
